# Initial kernel scaffold; baseline (speedup 1.0000x reference)
import jax
import jax.numpy as jnp
from jax.experimental import pallas as pl


def kernel(x, edge_index, batch, W1, b1, W2, b2, W3, b3, Wl, bl):
    raise NotImplementedError("write your pallas kernel here")


# speedup vs baseline: 9.3816x; 9.3816x over previous
import jax
import jax.numpy as jnp
from jax import lax
from jax.experimental import pallas as pl
from jax.experimental.pallas import tpu as pltpu
from jax.experimental.pallas import tpu_sc as plsc

N = 10000
D = 128
HD = D // 2
G = 64
E = 320000

NC = 2
NS = 16
NW = NC * NS

CH = 128
NCHA = 160
NCHD = 80
EPAD = NS * NCHA * CH
NPAD = 10240
RPT = NPAD // NS

BN = 2000

_mesh = plsc.VectorSubcoreMesh(
    core_axis_name="c", subcore_axis_name="s", num_cores=NC, num_subcores=NS
)


def _deg_body(dst_hbm, out_hbm, dst_v, ones_v, zero_v, acc_sh):
  cid = lax.axis_index("c")
  sid = lax.axis_index("s")
  wid = cid * NS + sid

  def _fill(r, _):
    ones_v[r, pl.ds(0, 16)] = jnp.ones((16,), jnp.float32)
    zero_v[r, pl.ds(0, 16)] = jnp.zeros((16,), jnp.float32)
    return 0
  lax.fori_loop(0, CH, _fill, 0)

  for k in range(RPT // CH):
    pltpu.sync_copy(zero_v, acc_sh.at[pl.ds(sid * RPT + k * CH, CH)])
  plsc.subcore_barrier()

  pltpu.sync_copy(dst_hbm.at[wid], dst_v)

  def _step(j, _):
    pltpu.sync_copy(ones_v, acc_sh.at[dst_v.at[j]], add=True)
    return 0
  lax.fori_loop(0, NCHD, _step, 0)
  plsc.subcore_barrier()

  pltpu.sync_copy(
      acc_sh.at[pl.ds(sid * RPT, RPT)],
      out_hbm.at[cid, pl.ds(sid * RPT, RPT)],
  )


_deg_call = pl.kernel(
    _deg_body,
    out_type=jax.ShapeDtypeStruct((NC, NPAD, 16), jnp.float32),
    mesh=_mesh,
    compiler_params=pltpu.CompilerParams(use_tc_tiling_on_sc=False),
    scratch_types=[
        pltpu.VMEM((NCHD, CH), jnp.int32),
        pltpu.VMEM((CH, 16), jnp.float32),
        pltpu.VMEM((CH, 16), jnp.float32),
        pltpu.VMEM_SHARED((NPAD, 16), jnp.float32),
    ],
)


def _agg_body(g_hbm, src_hbm, dst_hbm, out_hbm, src_v, dst_v, rows_v, zero_v,
              acc_sh):
  cid = lax.axis_index("c")
  sid = lax.axis_index("s")

  def _fill(r, _):
    for c in range(HD // 16):
      zero_v[r, pl.ds(c * 16, 16)] = jnp.zeros((16,), jnp.float32)
    return 0
  lax.fori_loop(0, CH, _fill, 0)

  for k in range(RPT // CH):
    pltpu.sync_copy(zero_v, acc_sh.at[pl.ds(sid * RPT + k * CH, CH)])
  plsc.subcore_barrier()

  pltpu.sync_copy(src_hbm.at[cid, sid], src_v)
  pltpu.sync_copy(dst_hbm.at[sid], dst_v)

  def _step(j, _):
    pltpu.sync_copy(g_hbm.at[src_v.at[j]], rows_v)
    pltpu.sync_copy(rows_v, acc_sh.at[dst_v.at[j]], add=True)
    return 0
  lax.fori_loop(0, NCHA, _step, 0)
  plsc.subcore_barrier()

  pltpu.sync_copy(
      acc_sh.at[pl.ds(sid * RPT, RPT)],
      out_hbm.at[cid, pl.ds(sid * RPT, RPT)],
  )


_agg_call = pl.kernel(
    _agg_body,
    out_type=jax.ShapeDtypeStruct((NC, NPAD, HD), jnp.float32),
    mesh=_mesh,
    compiler_params=pltpu.CompilerParams(use_tc_tiling_on_sc=False),
    scratch_types=[
        pltpu.VMEM((NCHA, CH), jnp.int32),
        pltpu.VMEM((NCHA, CH), jnp.int32),
        pltpu.VMEM((CH, HD), jnp.float32),
        pltpu.VMEM((CH, HD), jnp.float32),
        pltpu.VMEM_SHARED((NPAD, HD), jnp.float32),
    ],
)


def _dinv_block(degp):
  deg = 1.0 + degp[0, :, 0] + degp[1, :, 0]
  return lax.rsqrt(deg)[:, None]


def _split_halves(g_ref, v):
  g_ref[0] = v[:, :HD]
  g_ref[1] = v[:, HD:]


def _k1_body(x_ref, w_ref, degp_ref, g_ref):
  dinv = _dinv_block(degp_ref[...])
  h = jnp.dot(x_ref[...], w_ref[...], preferred_element_type=jnp.float32)
  _split_halves(g_ref, h * dinv)


_k1_call = pl.pallas_call(
    _k1_body,
    grid=(N // BN,),
    in_specs=[
        pl.BlockSpec((BN, D), lambda i: (i, 0)),
        pl.BlockSpec((D, D), lambda i: (0, 0)),
        pl.BlockSpec((NC, BN, 16), lambda i: (0, i, 0)),
    ],
    out_specs=pl.BlockSpec((2, BN, HD), lambda i: (0, i, 0)),
    out_shape=jax.ShapeDtypeStruct((2, N, HD), jnp.float32),
)


def _k2_body(p_ref, gprev_ref, degp_ref, b_ref, w_ref, gout_ref):
  dinv = _dinv_block(degp_ref[...])
  s = jnp.concatenate(
      [p_ref[0] + gprev_ref[0], p_ref[1] + gprev_ref[1]], axis=1
  )
  h = jnp.maximum(s * dinv + b_ref[...], 0.0)
  hw = jnp.dot(h, w_ref[...], preferred_element_type=jnp.float32) * dinv
  _split_halves(gout_ref, hw)


_k2_call = pl.pallas_call(
    _k2_body,
    grid=(N // BN,),
    in_specs=[
        pl.BlockSpec((NC, BN, HD), lambda i: (0, i, 0)),
        pl.BlockSpec((2, BN, HD), lambda i: (0, i, 0)),
        pl.BlockSpec((NC, BN, 16), lambda i: (0, i, 0)),
        pl.BlockSpec((1, D), lambda i: (0, 0)),
        pl.BlockSpec((D, D), lambda i: (0, 0)),
    ],
    out_specs=pl.BlockSpec((2, BN, HD), lambda i: (0, i, 0)),
    out_shape=jax.ShapeDtypeStruct((2, N, HD), jnp.float32),
)


def _k4_body(p_ref, g3_ref, degp_ref, b_ref, batch_ref, wl_ref, bl_ref,
             out_ref, seg_acc, cnt_acc):
  i = pl.program_id(0)

  @pl.when(i == 0)
  def _init():
    seg_acc[...] = jnp.zeros_like(seg_acc)
    cnt_acc[...] = jnp.zeros_like(cnt_acc)

  dinv = _dinv_block(degp_ref[...])
  s = jnp.concatenate(
      [p_ref[0] + g3_ref[0], p_ref[1] + g3_ref[1]], axis=1
  )
  h3 = s * dinv + b_ref[...]

  gids = lax.broadcasted_iota(jnp.int32, (1, G), 1)
  oh = (batch_ref[...] == gids).astype(jnp.float32)
  seg_acc[...] += lax.dot_general(
      oh, h3, (((0,), (0,)), ((), ())), preferred_element_type=jnp.float32
  )
  cnt_acc[...] += jnp.sum(oh, axis=0)[:, None]

  pooled = seg_acc[...] / jnp.maximum(cnt_acc[...], 1.0)
  out_ref[...] = (
      jnp.dot(pooled, wl_ref[...], preferred_element_type=jnp.float32)
      + bl_ref[...]
  )


_k4_call = pl.pallas_call(
    _k4_body,
    grid=(N // BN,),
    in_specs=[
        pl.BlockSpec((NC, BN, HD), lambda i: (0, i, 0)),
        pl.BlockSpec((2, BN, HD), lambda i: (0, i, 0)),
        pl.BlockSpec((NC, BN, 16), lambda i: (0, i, 0)),
        pl.BlockSpec((1, D), lambda i: (0, 0)),
        pl.BlockSpec((BN, 1), lambda i: (i, 0)),
        pl.BlockSpec((D, 3), lambda i: (0, 0)),
        pl.BlockSpec((1, 3), lambda i: (0, 0)),
    ],
    out_specs=pl.BlockSpec((G, 3), lambda i: (0, 0)),
    out_shape=jax.ShapeDtypeStruct((G, 3), jnp.float32),
    scratch_shapes=[
        pltpu.VMEM((G, D), jnp.float32),
        pltpu.VMEM((G, 1), jnp.float32),
    ],
)


@jax.jit
def kernel(x, edge_index, batch, W1, b1, W2, b2, W3, b3, Wl, bl):
  src = edge_index[0].astype(jnp.int32)
  dst = edge_index[1].astype(jnp.int32)
  npad_e = EPAD - E
  srcp = jnp.concatenate([src, jnp.zeros((npad_e,), jnp.int32)])
  dstp = jnp.concatenate([dst, jnp.full((npad_e,), N, jnp.int32)])
  src_a = srcp.reshape(NS, NCHA, CH)
  src4 = jnp.stack([src_a, src_a + N])
  dst_a = dstp.reshape(NS, NCHA, CH)
  dst_d = dstp.reshape(NW, NCHD, CH)

  b1r = b1.reshape(1, D)
  b2r = b2.reshape(1, D)
  b3r = b3.reshape(1, D)
  blr = bl.reshape(1, 3)
  batch2 = batch.astype(jnp.int32).reshape(N, 1)

  degp = _deg_call(dst_d)
  g1 = _k1_call(x, W1, degp)
  p1 = _agg_call(g1.reshape(2 * N, HD), src4, dst_a)
  g2 = _k2_call(p1, g1, degp, b1r, W2)
  p2 = _agg_call(g2.reshape(2 * N, HD), src4, dst_a)
  g3 = _k2_call(p2, g2, degp, b2r, W3)
  p3 = _agg_call(g3.reshape(2 * N, HD), src4, dst_a)
  return _k4_call(p3, g3, degp, b3r, batch2, Wl, blr)


# speedup vs baseline: 12.5911x; 1.3421x over previous
import jax
import jax.numpy as jnp
from jax import lax
from jax.experimental import pallas as pl
from jax.experimental.pallas import tpu as pltpu
from jax.experimental.pallas import tpu_sc as plsc

N = 10000
D = 128
HD = D // 2
G = 64
E = 320000

NC = 2
NS = 16
NW = NC * NS

CH = 128
NCHA = 160
NCHD = 80
EPAD = NS * NCHA * CH
NPAD = 10240
RPT = NPAD // NS

BN = 2000

_mesh = plsc.VectorSubcoreMesh(
    core_axis_name="c", subcore_axis_name="s", num_cores=NC, num_subcores=NS
)


def _deg_body(dst_hbm, out_hbm, dst_v, ones_v, zero_v, acc_sh):
  cid = lax.axis_index("c")
  sid = lax.axis_index("s")
  wid = cid * NS + sid

  def _fill(r, _):
    ones_v[r, pl.ds(0, 16)] = jnp.ones((16,), jnp.float32)
    zero_v[r, pl.ds(0, 16)] = jnp.zeros((16,), jnp.float32)
    return 0
  lax.fori_loop(0, CH, _fill, 0)

  for k in range(RPT // CH):
    pltpu.sync_copy(zero_v, acc_sh.at[pl.ds(sid * RPT + k * CH, CH)])
  plsc.subcore_barrier()

  pltpu.sync_copy(dst_hbm.at[wid], dst_v)

  def _step(j, _):
    pltpu.sync_copy(ones_v, acc_sh.at[dst_v.at[j]], add=True)
    return 0
  lax.fori_loop(0, NCHD, _step, 0)
  plsc.subcore_barrier()

  pltpu.sync_copy(
      acc_sh.at[pl.ds(sid * RPT, RPT)],
      out_hbm.at[cid, pl.ds(sid * RPT, RPT)],
  )


_deg_call = pl.kernel(
    _deg_body,
    out_type=jax.ShapeDtypeStruct((NC, NPAD, 16), jnp.float32),
    mesh=_mesh,
    compiler_params=pltpu.CompilerParams(use_tc_tiling_on_sc=False),
    scratch_types=[
        pltpu.VMEM((NCHD, CH), jnp.int32),
        pltpu.VMEM((CH, 16), jnp.float32),
        pltpu.VMEM((CH, 16), jnp.float32),
        pltpu.VMEM_SHARED((NPAD, 16), jnp.float32),
    ],
)


NBUF = 4
NGRP = NCHA // NBUF


def _agg_body(g_hbm, src_hbm, dst_hbm, out_hbm, src_v, dst_v, rows_v, zero_v,
              acc_sh, isems, gsems):
  cid = lax.axis_index("c")
  sid = lax.axis_index("s")

  idx_src = pltpu.async_copy(src_hbm.at[cid, sid], src_v, isems.at[0])
  idx_dst = pltpu.async_copy(dst_hbm.at[sid], dst_v, isems.at[1])

  def _fill(r, _):
    for c in range(HD // 16):
      zero_v[r, pl.ds(c * 16, 16)] = jnp.zeros((16,), jnp.float32)
    return 0
  lax.fori_loop(0, CH, _fill, 0)

  for k in range(RPT // CH):
    pltpu.sync_copy(zero_v, acc_sh.at[pl.ds(sid * RPT + k * CH, CH)])
  idx_src.wait()
  idx_dst.wait()
  plsc.subcore_barrier()

  for b in range(NBUF):
    pltpu.async_copy(g_hbm.at[src_v.at[b]], rows_v.at[b], gsems.at[b])

  def _group(j0, _):
    for b in range(NBUF):
      j = j0 * NBUF + b
      pltpu.make_async_copy(
          g_hbm.at[src_v.at[j]], rows_v.at[b], gsems.at[b]
      ).wait()
      pltpu.sync_copy(rows_v.at[b], acc_sh.at[dst_v.at[j]], add=True)

      @pl.when(j0 < NGRP - 1)
      def _refill():
        pltpu.async_copy(
            g_hbm.at[src_v.at[j + NBUF]], rows_v.at[b], gsems.at[b]
        )
    return 0
  lax.fori_loop(0, NGRP, _group, 0)
  plsc.subcore_barrier()

  pltpu.sync_copy(
      acc_sh.at[pl.ds(sid * RPT, RPT)],
      out_hbm.at[cid, pl.ds(sid * RPT, RPT)],
  )


_agg_call = pl.kernel(
    _agg_body,
    out_type=jax.ShapeDtypeStruct((NC, NPAD, HD), jnp.float32),
    mesh=_mesh,
    compiler_params=pltpu.CompilerParams(use_tc_tiling_on_sc=False),
    scratch_types=[
        pltpu.VMEM((NCHA, CH), jnp.int32),
        pltpu.VMEM((NCHA, CH), jnp.int32),
        pltpu.VMEM((NBUF, CH, HD), jnp.float32),
        pltpu.VMEM((CH, HD), jnp.float32),
        pltpu.VMEM_SHARED((NPAD, HD), jnp.float32),
        pltpu.SemaphoreType.DMA((2,)),
        pltpu.SemaphoreType.DMA((NBUF,)),
    ],
)


def _dinv_block(degp):
  deg = 1.0 + degp[0, :, 0] + degp[1, :, 0]
  return lax.rsqrt(deg)[:, None]


def _split_halves(g_ref, v):
  g_ref[0] = v[:, :HD]
  g_ref[1] = v[:, HD:]


def _k1_body(x_ref, w_ref, degp_ref, g_ref):
  dinv = _dinv_block(degp_ref[...])
  h = jnp.dot(x_ref[...], w_ref[...], preferred_element_type=jnp.float32)
  _split_halves(g_ref, h * dinv)


_k1_call = pl.pallas_call(
    _k1_body,
    grid=(N // BN,),
    in_specs=[
        pl.BlockSpec((BN, D), lambda i: (i, 0)),
        pl.BlockSpec((D, D), lambda i: (0, 0)),
        pl.BlockSpec((NC, BN, 16), lambda i: (0, i, 0)),
    ],
    out_specs=pl.BlockSpec((2, BN, HD), lambda i: (0, i, 0)),
    out_shape=jax.ShapeDtypeStruct((2, N, HD), jnp.float32),
)


def _k2_body(p_ref, gprev_ref, degp_ref, b_ref, w_ref, gout_ref):
  dinv = _dinv_block(degp_ref[...])
  s = jnp.concatenate(
      [p_ref[0] + gprev_ref[0], p_ref[1] + gprev_ref[1]], axis=1
  )
  h = jnp.maximum(s * dinv + b_ref[...], 0.0)
  hw = jnp.dot(h, w_ref[...], preferred_element_type=jnp.float32) * dinv
  _split_halves(gout_ref, hw)


_k2_call = pl.pallas_call(
    _k2_body,
    grid=(N // BN,),
    in_specs=[
        pl.BlockSpec((NC, BN, HD), lambda i: (0, i, 0)),
        pl.BlockSpec((2, BN, HD), lambda i: (0, i, 0)),
        pl.BlockSpec((NC, BN, 16), lambda i: (0, i, 0)),
        pl.BlockSpec((1, D), lambda i: (0, 0)),
        pl.BlockSpec((D, D), lambda i: (0, 0)),
    ],
    out_specs=pl.BlockSpec((2, BN, HD), lambda i: (0, i, 0)),
    out_shape=jax.ShapeDtypeStruct((2, N, HD), jnp.float32),
)


def _k4_body(p_ref, g3_ref, degp_ref, b_ref, batch_ref, wl_ref, bl_ref,
             out_ref, seg_acc, cnt_acc):
  i = pl.program_id(0)

  @pl.when(i == 0)
  def _init():
    seg_acc[...] = jnp.zeros_like(seg_acc)
    cnt_acc[...] = jnp.zeros_like(cnt_acc)

  dinv = _dinv_block(degp_ref[...])
  s = jnp.concatenate(
      [p_ref[0] + g3_ref[0], p_ref[1] + g3_ref[1]], axis=1
  )
  h3 = s * dinv + b_ref[...]

  gids = lax.broadcasted_iota(jnp.int32, (1, G), 1)
  oh = (batch_ref[...] == gids).astype(jnp.float32)
  seg_acc[...] += lax.dot_general(
      oh, h3, (((0,), (0,)), ((), ())), preferred_element_type=jnp.float32
  )
  cnt_acc[...] += jnp.sum(oh, axis=0)[:, None]

  pooled = seg_acc[...] / jnp.maximum(cnt_acc[...], 1.0)
  out_ref[...] = (
      jnp.dot(pooled, wl_ref[...], preferred_element_type=jnp.float32)
      + bl_ref[...]
  )


_k4_call = pl.pallas_call(
    _k4_body,
    grid=(N // BN,),
    in_specs=[
        pl.BlockSpec((NC, BN, HD), lambda i: (0, i, 0)),
        pl.BlockSpec((2, BN, HD), lambda i: (0, i, 0)),
        pl.BlockSpec((NC, BN, 16), lambda i: (0, i, 0)),
        pl.BlockSpec((1, D), lambda i: (0, 0)),
        pl.BlockSpec((BN, 1), lambda i: (i, 0)),
        pl.BlockSpec((D, 3), lambda i: (0, 0)),
        pl.BlockSpec((1, 3), lambda i: (0, 0)),
    ],
    out_specs=pl.BlockSpec((G, 3), lambda i: (0, 0)),
    out_shape=jax.ShapeDtypeStruct((G, 3), jnp.float32),
    scratch_shapes=[
        pltpu.VMEM((G, D), jnp.float32),
        pltpu.VMEM((G, 1), jnp.float32),
    ],
)


@jax.jit
def kernel(x, edge_index, batch, W1, b1, W2, b2, W3, b3, Wl, bl):
  src = edge_index[0].astype(jnp.int32)
  dst = edge_index[1].astype(jnp.int32)
  npad_e = EPAD - E
  srcp = jnp.concatenate([src, jnp.zeros((npad_e,), jnp.int32)])
  dstp = jnp.concatenate([dst, jnp.full((npad_e,), N, jnp.int32)])
  src_a = srcp.reshape(NS, NCHA, CH)
  src4 = jnp.stack([src_a, src_a + N])
  dst_a = dstp.reshape(NS, NCHA, CH)
  dst_d = dstp.reshape(NW, NCHD, CH)

  b1r = b1.reshape(1, D)
  b2r = b2.reshape(1, D)
  b3r = b3.reshape(1, D)
  blr = bl.reshape(1, 3)
  batch2 = batch.astype(jnp.int32).reshape(N, 1)

  degp = _deg_call(dst_d)
  g1 = _k1_call(x, W1, degp)
  p1 = _agg_call(g1.reshape(2 * N, HD), src4, dst_a)
  g2 = _k2_call(p1, g1, degp, b1r, W2)
  p2 = _agg_call(g2.reshape(2 * N, HD), src4, dst_a)
  g3 = _k2_call(p2, g2, degp, b2r, W3)
  p3 = _agg_call(g3.reshape(2 * N, HD), src4, dst_a)
  return _k4_call(p3, g3, degp, b3r, batch2, Wl, blr)
